# Initial kernel scaffold; baseline (speedup 1.0000x reference)
import jax
import jax.numpy as jnp
from jax.experimental import pallas as pl


def kernel(xyz, colors, params):
    raise NotImplementedError("write your pallas kernel here")


# speedup vs baseline: 8.7517x; 8.7517x over previous
import functools
import math

import jax
import jax.numpy as jnp
from jax import lax
from jax.experimental import pallas as pl
from jax.experimental.pallas import tpu as pltpu
from jax.experimental.pallas import tpu_sc as plsc

_F32 = jnp.float32
_BIG = 1e9


def _sqdist(src, dst):
    M = src.shape[0]
    N = dst.shape[1]
    a2 = (src[:, 0:1] * src[:, 0:1] + src[:, 1:2] * src[:, 1:2]
          + src[:, 2:3] * src[:, 2:3])
    b2 = (dst[0:1] * dst[0:1] + dst[1:2] * dst[1:2] + dst[2:3] * dst[2:3])
    sb = src.astype(jnp.bfloat16).astype(_F32)
    db = dst.astype(jnp.bfloat16).astype(_F32)

    def outer(c):
        return (jnp.broadcast_to(sb[:, c:c + 1], (M, N))
                * jnp.broadcast_to(db[c:c + 1, :], (M, N)))

    cross = outer(0) + outer(1) + outer(2)
    return a2 + b2 - 2.0 * cross


def _bdot(a, b):
    return jax.lax.dot_general(
        a.astype(jnp.bfloat16), b.astype(jnp.bfloat16),
        (((1,), (0,)), ((), ())), preferred_element_type=_F32)


_NC = 2
_NW = 32


def _sc_gather(table, idx):
    V, D = table.shape
    R = idx.shape[0]
    assert R % _NW == 0 and D % 16 == 0
    r = R // _NW
    c = 8
    for cand in range(128, 7, -8):
        if r % cand == 0 and cand * D * 4 <= 384 * 1024:
            c = cand
            break
    assert c >= 8 and r % c == 0
    nit = r // c
    mesh = plsc.VectorSubcoreMesh(core_axis_name="c", subcore_axis_name="s")

    @functools.partial(
        pl.kernel,
        mesh=mesh,
        compiler_params=pltpu.CompilerParams(use_tc_tiling_on_sc=False),
        out_type=jax.ShapeDtypeStruct((R, D), _F32),
        scratch_types=[
            pltpu.VMEM((c,), jnp.int32),
            pltpu.VMEM((c, D), _F32),
            pltpu.SemaphoreType.DMA,
        ],
    )
    def k(table_hbm, idx_hbm, out_hbm, idx_v, rows_v, sem):
        wid = lax.axis_index("s") * _NC + lax.axis_index("c")
        base = wid * r

        def step(j, carry):
            off = base + j * c
            pltpu.sync_copy(idx_hbm.at[pl.ds(off, c)], idx_v)
            pltpu.async_copy(table_hbm.at[idx_v], rows_v, sem).wait()
            pltpu.sync_copy(rows_v, out_hbm.at[pl.ds(off, c)])
            return carry

        lax.fori_loop(0, nit, step, 0)

    return k(table, idx)


def _gather_rows(table, idx):
    return _sc_gather(table, idx)


def _fps_call(xs, ys, zs, npoint):
    B, N = xs.shape

    def body(xs_ref, ys_ref, zs_ref, nx_ref, ny_ref, nz_ref):
        iota = lax.broadcasted_iota(jnp.int32, (B, N), 1)
        iota_s = lax.broadcasted_iota(jnp.int32, (B, npoint), 1)
        X = xs_ref[...]
        Y = ys_ref[...]
        Z = zs_ref[...]

        def step(i, carry):
            dist, far, ax, ay, az = carry
            mask = iota == far
            cx = jnp.sum(jnp.where(mask, X, 0.0), axis=1, keepdims=True)
            cy = jnp.sum(jnp.where(mask, Y, 0.0), axis=1, keepdims=True)
            cz = jnp.sum(jnp.where(mask, Z, 0.0), axis=1, keepdims=True)
            slot = iota_s == i
            ax = jnp.where(slot, cx, ax)
            ay = jnp.where(slot, cy, ay)
            az = jnp.where(slot, cz, az)
            d = (X - cx) ** 2 + (Y - cy) ** 2 + (Z - cz) ** 2
            dist = jnp.minimum(dist, d)
            m = jnp.max(dist, axis=1, keepdims=True)
            far = jnp.min(jnp.where(dist == m, iota, N), axis=1, keepdims=True)
            return dist, far, ax, ay, az

        zc = jnp.zeros((B, npoint), _F32)
        dist, far, ax, ay, az = lax.fori_loop(
            0,
            npoint,
            step,
            (jnp.full((B, N), 1e10, _F32), jnp.zeros((B, 1), jnp.int32),
             zc, zc, zc),
        )
        nx_ref[...] = ax
        ny_ref[...] = ay
        nz_ref[...] = az

    return pl.pallas_call(
        body,
        out_shape=[jax.ShapeDtypeStruct((B, npoint), _F32)] * 3,
    )(xs, ys, zs)


def _ballquery_call(new_rows, xyz_cols, rad1, rad2, K1, K2):
    B, S, _ = new_rows.shape
    N = xyz_cols.shape[2]
    Ts = min(64, S)
    nb = S // Ts
    r1sq = float(rad1 * rad1)
    r2sq = float(rad2 * rad2)

    def body(nr_ref, xc_ref, o1_ref, o2_ref):
        b = pl.program_id(0)
        nr = nr_ref[0]
        xc = xc_ref[0]
        d = _sqdist(nr, xc)
        iota = lax.broadcasted_iota(jnp.int32, (Ts, N), 1).astype(_F32)
        off = b * N
        for rsq, K, o_ref in ((r1sq, K1, o1_ref), (r2sq, K2, o2_ref)):
            v = jnp.where(d <= rsq, iota, jnp.float32(_BIG))
            cols = []
            first = None
            for k in range(K):
                m = jnp.min(v, axis=1, keepdims=True)
                if k == 0:
                    first = jnp.minimum(m, jnp.float32(N))
                    cols.append(first)
                else:
                    cols.append(jnp.where(m >= N, first, m))
                v = jnp.where(v == m, jnp.float32(_BIG), v)
            sel = jnp.minimum(jnp.concatenate(cols, axis=1),
                              jnp.float32(N - 1))
            o_ref[0] = sel.astype(jnp.int32) + off

    return pl.pallas_call(
        body,
        grid=(B, nb),
        in_specs=[
            pl.BlockSpec((1, Ts, 3), lambda b, s: (b, s, 0)),
            pl.BlockSpec((1, 3, N), lambda b, s: (b, 0, 0)),
        ],
        out_specs=[
            pl.BlockSpec((1, Ts, K1), lambda b, s: (b, s, 0)),
            pl.BlockSpec((1, Ts, K2), lambda b, s: (b, s, 0)),
        ],
        out_shape=[
            jax.ShapeDtypeStruct((B, S, K1), jnp.int32),
            jax.ShapeDtypeStruct((B, S, K2), jnp.int32),
        ],
    )(new_rows, xyz_cols)


def _knn3_call(x1_rows, x2_cols):
    B, N, _ = x1_rows.shape
    S = x2_cols.shape[2]
    Tn = min(512, N)
    nb = N // Tn

    def body(x1_ref, x2_ref, o_ref):
        b = pl.program_id(0)
        x1 = x1_ref[0]
        x2 = x2_ref[0]
        d = _sqdist(x1, x2)
        iota = lax.broadcasted_iota(jnp.int32, (Tn, S), 1).astype(_F32)
        v = d
        idxs, vals = [], []
        for _ in range(3):
            m = jnp.min(v, axis=1, keepdims=True)
            ix = jnp.min(
                jnp.where(v == m, iota, jnp.float32(_BIG)), axis=1, keepdims=True
            )
            idxs.append(ix)
            vals.append(m)
            v = jnp.where(iota == ix, jnp.float32(_BIG), v)
        r0 = 1.0 / (vals[0] + 1e-8)
        r1 = 1.0 / (vals[1] + 1e-8)
        r2 = 1.0 / (vals[2] + 1e-8)
        tot = r0 + r1 + r2
        off = (b * S).astype(_F32)
        zero = jnp.zeros_like(r0)
        o_ref[0, 0] = jnp.concatenate(
            [idxs[0] + off, idxs[1] + off, idxs[2] + off,
             r0 / tot, r1 / tot, r2 / tot, zero, zero],
            axis=1,
        )

    return pl.pallas_call(
        body,
        grid=(B, nb),
        in_specs=[
            pl.BlockSpec((1, Tn, 3), lambda b, t: (b, t, 0)),
            pl.BlockSpec((1, 3, S), lambda b, t: (b, 0, 0)),
        ],
        out_specs=pl.BlockSpec((1, 1, Tn, 8), lambda b, t: (b, t, 0, 0)),
        out_shape=jax.ShapeDtypeStruct((B, nb, Tn, 8), _F32),
    )(x1_rows, x2_cols)


def _row_tile(R):
    return min(512, R)


def _conv_sa_first(x, ne, wt, wxt, b):
    R, Dp = x.shape
    O = wt.shape[1]
    T = _row_tile(R)
    nb = R // T

    def body(x_ref, ne_ref, wt_ref, wxt_ref, b_ref, z_ref, s_ref, q_ref, acc):
        i = pl.program_id(0)

        @pl.when(i == 0)
        def _():
            acc[...] = jnp.zeros_like(acc)

        z = (
            _bdot(x_ref[...], wt_ref[...])
            - _bdot(ne_ref[...], wxt_ref[...])
            + b_ref[...]
        )
        z_ref[...] = z
        acc[0:1, :] = acc[0:1, :] + jnp.sum(z, axis=0, keepdims=True)
        acc[1:2, :] = acc[1:2, :] + jnp.sum(z * z, axis=0, keepdims=True)

        @pl.when(i == nb - 1)
        def _():
            s_ref[...] = acc[0:1, :]
            q_ref[...] = acc[1:2, :]

    return pl.pallas_call(
        body,
        grid=(nb,),
        in_specs=[
            pl.BlockSpec((T, Dp), lambda i: (i, 0)),
            pl.BlockSpec((T, 3), lambda i: (i, 0)),
            pl.BlockSpec((Dp, O), lambda i: (0, 0)),
            pl.BlockSpec((3, O), lambda i: (0, 0)),
            pl.BlockSpec((1, O), lambda i: (0, 0)),
        ],
        out_specs=[
            pl.BlockSpec((T, O), lambda i: (i, 0)),
            pl.BlockSpec((1, O), lambda i: (0, 0)),
            pl.BlockSpec((1, O), lambda i: (0, 0)),
        ],
        out_shape=[
            jax.ShapeDtypeStruct((R, O), _F32),
            jax.ShapeDtypeStruct((1, O), _F32),
            jax.ShapeDtypeStruct((1, O), _F32),
        ],
        scratch_shapes=[pltpu.VMEM((8, O), _F32)],
    )(x, ne, wt, wxt, b)


def _conv_mid(zp, a, bb, wt, b):
    R, C = zp.shape
    O = wt.shape[1]
    T = _row_tile(R)
    nb = R // T

    def body(zp_ref, a_ref, bb_ref, wt_ref, b_ref, z_ref, s_ref, q_ref, acc):
        i = pl.program_id(0)

        @pl.when(i == 0)
        def _():
            acc[...] = jnp.zeros_like(acc)

        y = jnp.maximum(zp_ref[...] * a_ref[...] + bb_ref[...], 0.0)
        z = _bdot(y, wt_ref[...]) + b_ref[...]
        z_ref[...] = z
        acc[0:1, :] = acc[0:1, :] + jnp.sum(z, axis=0, keepdims=True)
        acc[1:2, :] = acc[1:2, :] + jnp.sum(z * z, axis=0, keepdims=True)

        @pl.when(i == nb - 1)
        def _():
            s_ref[...] = acc[0:1, :]
            q_ref[...] = acc[1:2, :]

    return pl.pallas_call(
        body,
        grid=(nb,),
        in_specs=[
            pl.BlockSpec((T, C), lambda i: (i, 0)),
            pl.BlockSpec((1, C), lambda i: (0, 0)),
            pl.BlockSpec((1, C), lambda i: (0, 0)),
            pl.BlockSpec((C, O), lambda i: (0, 0)),
            pl.BlockSpec((1, O), lambda i: (0, 0)),
        ],
        out_specs=[
            pl.BlockSpec((T, O), lambda i: (i, 0)),
            pl.BlockSpec((1, O), lambda i: (0, 0)),
            pl.BlockSpec((1, O), lambda i: (0, 0)),
        ],
        out_shape=[
            jax.ShapeDtypeStruct((R, O), _F32),
            jax.ShapeDtypeStruct((1, O), _F32),
            jax.ShapeDtypeStruct((1, O), _F32),
        ],
        scratch_shapes=[pltpu.VMEM((8, O), _F32)],
    )(zp, a, bb, wt, b)


def _conv_fp_first(p1, g0, g1, g2, w3, wpt, wit, b):
    R, C2 = g0.shape
    O = wit.shape[1]
    T = _row_tile(R)
    nb = R // T
    has_p1 = p1 is not None

    def body(*refs):
        if has_p1:
            (p1_ref, g0_ref, g1_ref, g2_ref, w3_ref, wpt_ref, wit_ref, b_ref,
             z_ref, s_ref, q_ref, acc) = refs
        else:
            (g0_ref, g1_ref, g2_ref, w3_ref, wit_ref, b_ref,
             z_ref, s_ref, q_ref, acc) = refs
        i = pl.program_id(0)

        @pl.when(i == 0)
        def _():
            acc[...] = jnp.zeros_like(acc)

        w = w3_ref[...]
        interp = (
            g0_ref[...] * w[:, 0:1]
            + g1_ref[...] * w[:, 1:2]
            + g2_ref[...] * w[:, 2:3]
        )
        z = _bdot(interp, wit_ref[...]) + b_ref[...]
        if has_p1:
            z = z + _bdot(p1_ref[...], wpt_ref[...])
        z_ref[...] = z
        acc[0:1, :] = acc[0:1, :] + jnp.sum(z, axis=0, keepdims=True)
        acc[1:2, :] = acc[1:2, :] + jnp.sum(z * z, axis=0, keepdims=True)

        @pl.when(i == nb - 1)
        def _():
            s_ref[...] = acc[0:1, :]
            q_ref[...] = acc[1:2, :]

    in_specs = []
    args = []
    if has_p1:
        C1 = p1.shape[1]
        in_specs.append(pl.BlockSpec((T, C1), lambda i: (i, 0)))
        args.append(p1)
    in_specs += [
        pl.BlockSpec((T, C2), lambda i: (i, 0)),
        pl.BlockSpec((T, C2), lambda i: (i, 0)),
        pl.BlockSpec((T, C2), lambda i: (i, 0)),
        pl.BlockSpec((T, 3), lambda i: (i, 0)),
    ]
    args += [g0, g1, g2, w3]
    if has_p1:
        in_specs.append(pl.BlockSpec((C1, O), lambda i: (0, 0)))
        args.append(wpt)
    in_specs += [
        pl.BlockSpec((C2, O), lambda i: (0, 0)),
        pl.BlockSpec((1, O), lambda i: (0, 0)),
    ]
    args += [wit, b]

    return pl.pallas_call(
        body,
        grid=(nb,),
        in_specs=in_specs,
        out_specs=[
            pl.BlockSpec((T, O), lambda i: (i, 0)),
            pl.BlockSpec((1, O), lambda i: (0, 0)),
            pl.BlockSpec((1, O), lambda i: (0, 0)),
        ],
        out_shape=[
            jax.ShapeDtypeStruct((R, O), _F32),
            jax.ShapeDtypeStruct((1, O), _F32),
            jax.ShapeDtypeStruct((1, O), _F32),
        ],
        scratch_shapes=[pltpu.VMEM((8, O), _F32)],
    )(*args)


def _pool_max(z, a, bb, K):
    R, O = z.shape
    T = _row_tile(R)
    nb = R // T

    def body(z_ref, a_ref, bb_ref, o_ref):
        y = jnp.maximum(z_ref[...] * a_ref[...] + bb_ref[...], 0.0)
        o_ref[...] = jnp.max(y.reshape(T // K, K, O), axis=1)

    return pl.pallas_call(
        body,
        grid=(nb,),
        in_specs=[
            pl.BlockSpec((T, O), lambda i: (i, 0)),
            pl.BlockSpec((1, O), lambda i: (0, 0)),
            pl.BlockSpec((1, O), lambda i: (0, 0)),
        ],
        out_specs=pl.BlockSpec((T // K, O), lambda i: (i, 0)),
        out_shape=jax.ShapeDtypeStruct((R // K, O), _F32),
    )(z, a, bb)


def _bn_relu(z, a, bb):
    R, O = z.shape
    T = _row_tile(R)
    nb = R // T

    def body(z_ref, a_ref, bb_ref, o_ref):
        o_ref[...] = jnp.maximum(z_ref[...] * a_ref[...] + bb_ref[...], 0.0)

    return pl.pallas_call(
        body,
        grid=(nb,),
        in_specs=[
            pl.BlockSpec((T, O), lambda i: (i, 0)),
            pl.BlockSpec((1, O), lambda i: (0, 0)),
            pl.BlockSpec((1, O), lambda i: (0, 0)),
        ],
        out_specs=pl.BlockSpec((T, O), lambda i: (i, 0)),
        out_shape=jax.ShapeDtypeStruct((R, O), _F32),
    )(z, a, bb)


def _final_call(z, a, bb, w2t, b2):
    R, C = z.shape
    O = w2t.shape[1]
    T = _row_tile(R)
    nb = R // T

    def body(z_ref, a_ref, bb_ref, w_ref, b2_ref, o_ref):
        y = jnp.maximum(z_ref[...] * a_ref[...] + bb_ref[...], 0.0)
        lg = _bdot(y, w_ref[...]) + b2_ref[...]
        mx = jnp.max(lg, axis=1, keepdims=True)
        sh = lg - mx
        lse = jnp.log(jnp.sum(jnp.exp(sh), axis=1, keepdims=True))
        o_ref[...] = sh - lse

    return pl.pallas_call(
        body,
        grid=(nb,),
        in_specs=[
            pl.BlockSpec((T, C), lambda i: (i, 0)),
            pl.BlockSpec((1, C), lambda i: (0, 0)),
            pl.BlockSpec((1, C), lambda i: (0, 0)),
            pl.BlockSpec((C, O), lambda i: (0, 0)),
            pl.BlockSpec((1, O), lambda i: (0, 0)),
        ],
        out_specs=pl.BlockSpec((T, O), lambda i: (i, 0)),
        out_shape=jax.ShapeDtypeStruct((R, O), _F32),
    )(z, a, bb, w2t, b2)


def _finalize_stats(s, q, count, g, be):
    mean = s.reshape(-1) / count
    var = q.reshape(-1) / count - mean * mean
    inv = 1.0 / jnp.sqrt(var + 1e-5)
    a = g * inv
    bb = be - mean * a
    return a.reshape(1, -1), bb.reshape(1, -1)


def _pad_cols(x, to):
    pad = to - x.shape[1]
    if pad == 0:
        return x
    return jnp.concatenate([x, jnp.zeros((x.shape[0], pad), _F32)], axis=1)


def _sa_level(xyz_c, points_rows, npoint, radii, nsamples, branch_params):
    B = xyz_c.shape[0]
    N = xyz_c.shape[2]
    S = npoint
    C = points_rows.shape[1]
    D = C + 3
    Dp = math.ceil(D / 16) * 16

    nx, ny, nz = _fps_call(xyz_c[:, 0], xyz_c[:, 1], xyz_c[:, 2], S)
    new_c = jnp.stack([nx, ny, nz], axis=1)
    new_rows3 = jnp.stack([nx, ny, nz], axis=-1)
    new_rows = new_rows3.reshape(B * S, 3)

    gi1, gi2 = _ballquery_call(
        new_rows3, xyz_c, radii[0], radii[1], nsamples[0], nsamples[1]
    )

    xyz_rows = jnp.transpose(xyz_c, (0, 2, 1)).reshape(B * N, 3)
    table = _pad_cols(jnp.concatenate([points_rows, xyz_rows], axis=1), Dp)

    R1 = B * S * nsamples[0]
    R2 = B * S * nsamples[1]
    g_all = _gather_rows(
        table, jnp.concatenate([gi1.reshape(R1), gi2.reshape(R2)]))
    gathered = (g_all[:R1], g_all[R1:])

    outs = []
    for g, K, layers in ((gathered[0], nsamples[0], branch_params[0]),
                         (gathered[1], nsamples[1], branch_params[1])):
        R = B * S * K
        ne = jnp.broadcast_to(
            new_rows[:, None, :], (B * S, K, 3)).reshape(R, 3)
        W0, b0, g0, be0 = layers[0]
        wt0 = jnp.concatenate(
            [jnp.transpose(W0), jnp.zeros((Dp - D, W0.shape[0]), _F32)], axis=0
        )
        wxt0 = jnp.transpose(W0[:, C:C + 3])
        z, s, q = _conv_sa_first(g, ne, wt0, wxt0, b0.reshape(1, -1))
        a, bb = _finalize_stats(s, q, R, g0, be0)
        for (W, b, gg, be) in layers[1:]:
            z, s, q = _conv_mid(z, a, bb, jnp.transpose(W), b.reshape(1, -1))
            a, bb = _finalize_stats(s, q, R, gg, be)
        outs.append(_pool_max(z, a, bb, K))
    return new_c, new_rows, jnp.concatenate(outs, axis=1)


def _fp_level(x1_c, p1_rows, x2_c, p2_rows, layer_params, materialize=True):
    B = x1_c.shape[0]
    N = x1_c.shape[2]
    S = x2_c.shape[2]
    R = B * N

    x1_rows3 = jnp.transpose(x1_c, (0, 2, 1))
    packed = _knn3_call(x1_rows3, x2_c)
    arr = packed.reshape(R, 8)
    i0 = arr[:, 0].astype(jnp.int32)
    i1 = arr[:, 1].astype(jnp.int32)
    i2 = arr[:, 2].astype(jnp.int32)
    w3 = arr[:, 3:6]

    g_all = _gather_rows(p2_rows, jnp.concatenate([i0, i1, i2]))
    g0, g1, g2 = g_all[:R], g_all[R:2 * R], g_all[2 * R:]

    W0, b0, gg0, be0 = layer_params[0]
    C2 = p2_rows.shape[1]
    if p1_rows is not None:
        C1 = p1_rows.shape[1]
        wpt = jnp.transpose(W0[:, :C1])
        wit = jnp.transpose(W0[:, C1:])
    else:
        wpt = None
        wit = jnp.transpose(W0)
    z, s, q = _conv_fp_first(p1_rows, g0, g1, g2, w3, wpt, wit, b0.reshape(1, -1))
    a, bb = _finalize_stats(s, q, R, gg0, be0)
    for (W, b, gg, be) in layer_params[1:]:
        z, s, q = _conv_mid(z, a, bb, jnp.transpose(W), b.reshape(1, -1))
        a, bb = _finalize_stats(s, q, R, gg, be)
    if materialize:
        return _bn_relu(z, a, bb)
    return z, a, bb


def kernel(xyz, colors, params):
    del colors
    B, _, N0 = xyz.shape

    mean = jnp.mean(xyz, axis=2, keepdims=True)
    std = jnp.std(xyz, axis=2, keepdims=True, ddof=1)
    std = jnp.where(std == 0, 1e-8, std)
    xyzn = (xyz - mean) / std
    l0_rows = jnp.transpose(xyzn, (0, 2, 1)).reshape(B * N0, 3)

    l1_c, l1_rows, l1_p = _sa_level(
        xyzn, l0_rows, 1024, [0.05, 0.1], [16, 32], params['sa1'])
    l2_c, l2_rows, l2_p = _sa_level(
        l1_c, l1_p, 256, [0.1, 0.2], [16, 32], params['sa2'])
    l3_c, l3_rows, l3_p = _sa_level(
        l2_c, l2_p, 64, [0.2, 0.4], [16, 32], params['sa3'])
    l4_c, l4_rows, l4_p = _sa_level(
        l3_c, l3_p, 16, [0.4, 0.8], [16, 32], params['sa4'])

    l3_p = _fp_level(l3_c, l3_p, l4_c, l4_p, params['fp4'])
    l2_p = _fp_level(l2_c, l2_p, l3_c, l3_p, params['fp3'])
    l1_p = _fp_level(l1_c, l1_p, l2_c, l2_p, params['fp2'])
    z, a, bb = _fp_level(xyzn, None, l1_c, l1_p, params['fp1'],
                         materialize=False)

    W1, b1, g1, be1 = params['conv1']
    z, s, q = _conv_mid(z, a, bb, jnp.transpose(W1), b1.reshape(1, -1))
    a, bb = _finalize_stats(s, q, B * N0, g1, be1)

    W2, b2 = params['conv2']
    out = _final_call(z, a, bb, jnp.transpose(W2), b2.reshape(1, -1))
    return out.reshape(B, N0, W2.shape[0])


# speedup vs baseline: 8.8710x; 1.0136x over previous
import functools
import math

import jax
import jax.numpy as jnp
from jax import lax
from jax.experimental import pallas as pl
from jax.experimental.pallas import tpu as pltpu
from jax.experimental.pallas import tpu_sc as plsc

_F32 = jnp.float32
_BIG = 1e9


def _sqdist(src, dst):
    a2 = (src[:, 0:1] * src[:, 0:1] + src[:, 1:2] * src[:, 1:2]
          + src[:, 2:3] * src[:, 2:3])
    b2 = (dst[0:1] * dst[0:1] + dst[1:2] * dst[1:2] + dst[2:3] * dst[2:3])
    cross = jax.lax.dot_general(
        src.astype(jnp.bfloat16), dst.astype(jnp.bfloat16),
        (((1,), (0,)), ((), ())), preferred_element_type=_F32)
    return a2 + b2 - 2.0 * cross


def _bdot(a, b):
    return jax.lax.dot_general(
        a.astype(jnp.bfloat16), b.astype(jnp.bfloat16),
        (((1,), (0,)), ((), ())), preferred_element_type=_F32)


_NC = 2
_NW = 32


def _sc_gather(table, idx):
    V, D = table.shape
    R = idx.shape[0]
    assert R % _NW == 0 and D % 16 == 0
    r = R // _NW
    c = 8
    for cand in range(128, 7, -8):
        if r % cand == 0 and cand * D * 4 <= 384 * 1024:
            c = cand
            break
    assert c >= 8 and r % c == 0
    nit = r // c
    mesh = plsc.VectorSubcoreMesh(core_axis_name="c", subcore_axis_name="s")

    @functools.partial(
        pl.kernel,
        mesh=mesh,
        compiler_params=pltpu.CompilerParams(use_tc_tiling_on_sc=False),
        out_type=jax.ShapeDtypeStruct((R, D), _F32),
        scratch_types=[
            pltpu.VMEM((c,), jnp.int32),
            pltpu.VMEM((c, D), _F32),
            pltpu.SemaphoreType.DMA,
        ],
    )
    def k(table_hbm, idx_hbm, out_hbm, idx_v, rows_v, sem):
        wid = lax.axis_index("s") * _NC + lax.axis_index("c")
        base = wid * r

        def step(j, carry):
            off = base + j * c
            pltpu.sync_copy(idx_hbm.at[pl.ds(off, c)], idx_v)
            pltpu.async_copy(table_hbm.at[idx_v], rows_v, sem).wait()
            pltpu.sync_copy(rows_v, out_hbm.at[pl.ds(off, c)])
            return carry

        lax.fori_loop(0, nit, step, 0)

    return k(table, idx)


def _gather_rows(table, idx):
    return _sc_gather(table, idx)


def _fps_call(xs, ys, zs, npoint):
    B, N = xs.shape

    def body(xs_ref, ys_ref, zs_ref, nx_ref, ny_ref, nz_ref):
        iota = lax.broadcasted_iota(jnp.int32, (B, N), 1)
        iota_s = lax.broadcasted_iota(jnp.int32, (B, npoint), 1)
        X = xs_ref[...]
        Y = ys_ref[...]
        Z = zs_ref[...]

        def step(i, carry):
            dist, far, ax, ay, az = carry
            mask = iota == far
            cx = jnp.sum(jnp.where(mask, X, 0.0), axis=1, keepdims=True)
            cy = jnp.sum(jnp.where(mask, Y, 0.0), axis=1, keepdims=True)
            cz = jnp.sum(jnp.where(mask, Z, 0.0), axis=1, keepdims=True)
            slot = iota_s == i
            ax = jnp.where(slot, cx, ax)
            ay = jnp.where(slot, cy, ay)
            az = jnp.where(slot, cz, az)
            d = (X - cx) ** 2 + (Y - cy) ** 2 + (Z - cz) ** 2
            dist = jnp.minimum(dist, d)
            m = jnp.max(dist, axis=1, keepdims=True)
            far = jnp.min(jnp.where(dist == m, iota, N), axis=1, keepdims=True)
            return dist, far, ax, ay, az

        zc = jnp.zeros((B, npoint), _F32)
        dist, far, ax, ay, az = lax.fori_loop(
            0,
            npoint,
            step,
            (jnp.full((B, N), 1e10, _F32), jnp.zeros((B, 1), jnp.int32),
             zc, zc, zc),
        )
        nx_ref[...] = ax
        ny_ref[...] = ay
        nz_ref[...] = az

    return pl.pallas_call(
        body,
        out_shape=[jax.ShapeDtypeStruct((B, npoint), _F32)] * 3,
    )(xs, ys, zs)


def _ballquery_call(new_rows, xyz_cols, rad1, rad2, K1, K2):
    B, S, _ = new_rows.shape
    N = xyz_cols.shape[2]
    Ts = min(64, S)
    nb = S // Ts
    r1sq = float(rad1 * rad1)
    r2sq = float(rad2 * rad2)

    def body(nr_ref, xc_ref, o1_ref, o2_ref):
        b = pl.program_id(0)
        nr = nr_ref[0]
        xc = xc_ref[0]
        d = _sqdist(nr, xc)
        iota = lax.broadcasted_iota(jnp.int32, (Ts, N), 1).astype(_F32)
        off = b * N
        for rsq, K, o_ref in ((r1sq, K1, o1_ref), (r2sq, K2, o2_ref)):
            v = jnp.where(d <= rsq, iota, jnp.float32(_BIG))
            cols = []
            first = None
            for k in range(K):
                m = jnp.min(v, axis=1, keepdims=True)
                if k == 0:
                    first = jnp.minimum(m, jnp.float32(N))
                    cols.append(first)
                else:
                    cols.append(jnp.where(m >= N, first, m))
                v = jnp.where(v == m, jnp.float32(_BIG), v)
            sel = jnp.minimum(jnp.concatenate(cols, axis=1),
                              jnp.float32(N - 1))
            o_ref[0] = sel.astype(jnp.int32) + off

    return pl.pallas_call(
        body,
        grid=(B, nb),
        in_specs=[
            pl.BlockSpec((1, Ts, 3), lambda b, s: (b, s, 0)),
            pl.BlockSpec((1, 3, N), lambda b, s: (b, 0, 0)),
        ],
        out_specs=[
            pl.BlockSpec((1, Ts, K1), lambda b, s: (b, s, 0)),
            pl.BlockSpec((1, Ts, K2), lambda b, s: (b, s, 0)),
        ],
        out_shape=[
            jax.ShapeDtypeStruct((B, S, K1), jnp.int32),
            jax.ShapeDtypeStruct((B, S, K2), jnp.int32),
        ],
    )(new_rows, xyz_cols)


def _knn3_call(x1_rows, x2_cols):
    B, N, _ = x1_rows.shape
    S = x2_cols.shape[2]
    Tn = min(512, N)
    nb = N // Tn

    def body(x1_ref, x2_ref, o_ref):
        b = pl.program_id(0)
        x1 = x1_ref[0]
        x2 = x2_ref[0]
        d = _sqdist(x1, x2)
        iota = lax.broadcasted_iota(jnp.int32, (Tn, S), 1).astype(_F32)
        v = d
        idxs, vals = [], []
        for _ in range(3):
            m = jnp.min(v, axis=1, keepdims=True)
            ix = jnp.min(
                jnp.where(v == m, iota, jnp.float32(_BIG)), axis=1, keepdims=True
            )
            idxs.append(ix)
            vals.append(m)
            v = jnp.where(iota == ix, jnp.float32(_BIG), v)
        off = (b * S).astype(_F32)
        zero = jnp.zeros_like(vals[0])
        o_ref[0, 0] = jnp.concatenate(
            [idxs[0] + off, idxs[1] + off, idxs[2] + off,
             vals[0], vals[1], vals[2], zero, zero],
            axis=1,
        )

    return pl.pallas_call(
        body,
        grid=(B, nb),
        in_specs=[
            pl.BlockSpec((1, Tn, 3), lambda b, t: (b, t, 0)),
            pl.BlockSpec((1, 3, S), lambda b, t: (b, 0, 0)),
        ],
        out_specs=pl.BlockSpec((1, 1, Tn, 8), lambda b, t: (b, t, 0, 0)),
        out_shape=jax.ShapeDtypeStruct((B, nb, Tn, 8), _F32),
    )(x1_rows, x2_cols)


def _row_tile(R):
    return min(512, R)


def _conv_sa_first(x, ne, wt, b, c_pts):
    R, Dp = x.shape
    O = wt.shape[1]
    T = _row_tile(R)
    nb = R // T
    npad = Dp - c_pts - 3

    def body(x_ref, ne_ref, wt_ref, b_ref, z_ref, s_ref, q_ref, acc):
        i = pl.program_id(0)

        @pl.when(i == 0)
        def _():
            acc[...] = jnp.zeros_like(acc)

        x = x_ref[...]
        ne = ne_ref[...]
        C = c_pts
        xm = jnp.concatenate(
            [x[:, :C], x[:, C:C + 3] - ne, x[:, C + 3:]], axis=1)
        z = _bdot(xm, wt_ref[...]) + b_ref[...]
        z_ref[...] = z
        acc[0:1, :] = acc[0:1, :] + jnp.sum(z, axis=0, keepdims=True)
        acc[1:2, :] = acc[1:2, :] + jnp.sum(z * z, axis=0, keepdims=True)

        @pl.when(i == nb - 1)
        def _():
            s_ref[...] = acc[0:1, :]
            q_ref[...] = acc[1:2, :]

    return pl.pallas_call(
        body,
        grid=(nb,),
        in_specs=[
            pl.BlockSpec((T, Dp), lambda i: (i, 0)),
            pl.BlockSpec((T, 3), lambda i: (i, 0)),
            pl.BlockSpec((Dp, O), lambda i: (0, 0)),
            pl.BlockSpec((1, O), lambda i: (0, 0)),
        ],
        out_specs=[
            pl.BlockSpec((T, O), lambda i: (i, 0)),
            pl.BlockSpec((1, O), lambda i: (0, 0)),
            pl.BlockSpec((1, O), lambda i: (0, 0)),
        ],
        out_shape=[
            jax.ShapeDtypeStruct((R, O), _F32),
            jax.ShapeDtypeStruct((1, O), _F32),
            jax.ShapeDtypeStruct((1, O), _F32),
        ],
        scratch_shapes=[pltpu.VMEM((8, O), _F32)],
    )(x, ne, wt, b)


def _conv_mid(zp, a, bb, wt, b):
    R, C = zp.shape
    O = wt.shape[1]
    T = _row_tile(R)
    nb = R // T

    def body(zp_ref, a_ref, bb_ref, wt_ref, b_ref, z_ref, s_ref, q_ref, acc):
        i = pl.program_id(0)

        @pl.when(i == 0)
        def _():
            acc[...] = jnp.zeros_like(acc)

        y = jnp.maximum(zp_ref[...] * a_ref[...] + bb_ref[...], 0.0)
        z = _bdot(y, wt_ref[...]) + b_ref[...]
        z_ref[...] = z
        acc[0:1, :] = acc[0:1, :] + jnp.sum(z, axis=0, keepdims=True)
        acc[1:2, :] = acc[1:2, :] + jnp.sum(z * z, axis=0, keepdims=True)

        @pl.when(i == nb - 1)
        def _():
            s_ref[...] = acc[0:1, :]
            q_ref[...] = acc[1:2, :]

    return pl.pallas_call(
        body,
        grid=(nb,),
        in_specs=[
            pl.BlockSpec((T, C), lambda i: (i, 0)),
            pl.BlockSpec((1, C), lambda i: (0, 0)),
            pl.BlockSpec((1, C), lambda i: (0, 0)),
            pl.BlockSpec((C, O), lambda i: (0, 0)),
            pl.BlockSpec((1, O), lambda i: (0, 0)),
        ],
        out_specs=[
            pl.BlockSpec((T, O), lambda i: (i, 0)),
            pl.BlockSpec((1, O), lambda i: (0, 0)),
            pl.BlockSpec((1, O), lambda i: (0, 0)),
        ],
        out_shape=[
            jax.ShapeDtypeStruct((R, O), _F32),
            jax.ShapeDtypeStruct((1, O), _F32),
            jax.ShapeDtypeStruct((1, O), _F32),
        ],
        scratch_shapes=[pltpu.VMEM((8, O), _F32)],
    )(zp, a, bb, wt, b)


def _conv_fp_first(p1, g0, g1, g2, w3, wpt, wit, b):
    R, C2 = g0.shape
    O = wit.shape[1]
    T = _row_tile(R)
    nb = R // T
    has_p1 = p1 is not None

    def body(*refs):
        if has_p1:
            (p1_ref, g0_ref, g1_ref, g2_ref, w3_ref, wpt_ref, wit_ref, b_ref,
             z_ref, s_ref, q_ref, acc) = refs
        else:
            (g0_ref, g1_ref, g2_ref, w3_ref, wit_ref, b_ref,
             z_ref, s_ref, q_ref, acc) = refs
        i = pl.program_id(0)

        @pl.when(i == 0)
        def _():
            acc[...] = jnp.zeros_like(acc)

        w = w3_ref[...]
        interp = (
            g0_ref[...] * w[:, 0:1]
            + g1_ref[...] * w[:, 1:2]
            + g2_ref[...] * w[:, 2:3]
        )
        z = _bdot(interp, wit_ref[...]) + b_ref[...]
        if has_p1:
            z = z + _bdot(p1_ref[...], wpt_ref[...])
        z_ref[...] = z
        acc[0:1, :] = acc[0:1, :] + jnp.sum(z, axis=0, keepdims=True)
        acc[1:2, :] = acc[1:2, :] + jnp.sum(z * z, axis=0, keepdims=True)

        @pl.when(i == nb - 1)
        def _():
            s_ref[...] = acc[0:1, :]
            q_ref[...] = acc[1:2, :]

    in_specs = []
    args = []
    if has_p1:
        C1 = p1.shape[1]
        in_specs.append(pl.BlockSpec((T, C1), lambda i: (i, 0)))
        args.append(p1)
    in_specs += [
        pl.BlockSpec((T, C2), lambda i: (i, 0)),
        pl.BlockSpec((T, C2), lambda i: (i, 0)),
        pl.BlockSpec((T, C2), lambda i: (i, 0)),
        pl.BlockSpec((T, 3), lambda i: (i, 0)),
    ]
    args += [g0, g1, g2, w3]
    if has_p1:
        in_specs.append(pl.BlockSpec((C1, O), lambda i: (0, 0)))
        args.append(wpt)
    in_specs += [
        pl.BlockSpec((C2, O), lambda i: (0, 0)),
        pl.BlockSpec((1, O), lambda i: (0, 0)),
    ]
    args += [wit, b]

    return pl.pallas_call(
        body,
        grid=(nb,),
        in_specs=in_specs,
        out_specs=[
            pl.BlockSpec((T, O), lambda i: (i, 0)),
            pl.BlockSpec((1, O), lambda i: (0, 0)),
            pl.BlockSpec((1, O), lambda i: (0, 0)),
        ],
        out_shape=[
            jax.ShapeDtypeStruct((R, O), _F32),
            jax.ShapeDtypeStruct((1, O), _F32),
            jax.ShapeDtypeStruct((1, O), _F32),
        ],
        scratch_shapes=[pltpu.VMEM((8, O), _F32)],
    )(*args)


def _pool_max(z, a, bb, K):
    R, O = z.shape
    T = _row_tile(R)
    nb = R // T

    def body(z_ref, a_ref, bb_ref, o_ref):
        y = jnp.maximum(z_ref[...] * a_ref[...] + bb_ref[...], 0.0)
        o_ref[...] = jnp.max(y.reshape(T // K, K, O), axis=1)

    return pl.pallas_call(
        body,
        grid=(nb,),
        in_specs=[
            pl.BlockSpec((T, O), lambda i: (i, 0)),
            pl.BlockSpec((1, O), lambda i: (0, 0)),
            pl.BlockSpec((1, O), lambda i: (0, 0)),
        ],
        out_specs=pl.BlockSpec((T // K, O), lambda i: (i, 0)),
        out_shape=jax.ShapeDtypeStruct((R // K, O), _F32),
    )(z, a, bb)


def _bn_relu(z, a, bb):
    R, O = z.shape
    T = _row_tile(R)
    nb = R // T

    def body(z_ref, a_ref, bb_ref, o_ref):
        o_ref[...] = jnp.maximum(z_ref[...] * a_ref[...] + bb_ref[...], 0.0)

    return pl.pallas_call(
        body,
        grid=(nb,),
        in_specs=[
            pl.BlockSpec((T, O), lambda i: (i, 0)),
            pl.BlockSpec((1, O), lambda i: (0, 0)),
            pl.BlockSpec((1, O), lambda i: (0, 0)),
        ],
        out_specs=pl.BlockSpec((T, O), lambda i: (i, 0)),
        out_shape=jax.ShapeDtypeStruct((R, O), _F32),
    )(z, a, bb)


def _final_call(z, a, bb, w2t, b2):
    R, C = z.shape
    O = w2t.shape[1]
    T = _row_tile(R)
    nb = R // T

    def body(z_ref, a_ref, bb_ref, w_ref, b2_ref, o_ref):
        y = jnp.maximum(z_ref[...] * a_ref[...] + bb_ref[...], 0.0)
        lg = _bdot(y, w_ref[...]) + b2_ref[...]
        mx = jnp.max(lg, axis=1, keepdims=True)
        sh = lg - mx
        lse = jnp.log(jnp.sum(jnp.exp(sh), axis=1, keepdims=True))
        o_ref[...] = sh - lse

    return pl.pallas_call(
        body,
        grid=(nb,),
        in_specs=[
            pl.BlockSpec((T, C), lambda i: (i, 0)),
            pl.BlockSpec((1, C), lambda i: (0, 0)),
            pl.BlockSpec((1, C), lambda i: (0, 0)),
            pl.BlockSpec((C, O), lambda i: (0, 0)),
            pl.BlockSpec((1, O), lambda i: (0, 0)),
        ],
        out_specs=pl.BlockSpec((T, O), lambda i: (i, 0)),
        out_shape=jax.ShapeDtypeStruct((R, O), _F32),
    )(z, a, bb, w2t, b2)


def _finalize_stats(s, q, count, g, be):
    mean = s.reshape(-1) / count
    var = q.reshape(-1) / count - mean * mean
    inv = 1.0 / jnp.sqrt(var + 1e-5)
    a = g * inv
    bb = be - mean * a
    return a.reshape(1, -1), bb.reshape(1, -1)


def _pad_cols(x, to):
    pad = to - x.shape[1]
    if pad == 0:
        return x
    return jnp.concatenate([x, jnp.zeros((x.shape[0], pad), _F32)], axis=1)


def _sa_level(xyz_c, points_rows, npoint, radii, nsamples, branch_params):
    B = xyz_c.shape[0]
    N = xyz_c.shape[2]
    S = npoint
    C = points_rows.shape[1]
    D = C + 3
    Dp = math.ceil(D / 16) * 16

    nx, ny, nz = _fps_call(xyz_c[:, 0], xyz_c[:, 1], xyz_c[:, 2], S)
    new_c = jnp.stack([nx, ny, nz], axis=1)
    new_rows3 = jnp.stack([nx, ny, nz], axis=-1)
    new_rows = new_rows3.reshape(B * S, 3)

    gi1, gi2 = _ballquery_call(
        new_rows3, xyz_c, radii[0], radii[1], nsamples[0], nsamples[1]
    )

    xyz_rows = jnp.transpose(xyz_c, (0, 2, 1)).reshape(B * N, 3)
    table = _pad_cols(jnp.concatenate([points_rows, xyz_rows], axis=1), Dp)

    R1 = B * S * nsamples[0]
    R2 = B * S * nsamples[1]
    g_all = _gather_rows(
        table, jnp.concatenate([gi1.reshape(R1), gi2.reshape(R2)]))
    gathered = (g_all[:R1], g_all[R1:])

    outs = []
    for g, K, layers in ((gathered[0], nsamples[0], branch_params[0]),
                         (gathered[1], nsamples[1], branch_params[1])):
        R = B * S * K
        ne = jnp.broadcast_to(
            new_rows[:, None, :], (B * S, K, 3)).reshape(R, 3)
        W0, b0, g0, be0 = layers[0]
        wt0 = jnp.concatenate(
            [jnp.transpose(W0), jnp.zeros((Dp - D, W0.shape[0]), _F32)], axis=0
        )
        z, s, q = _conv_sa_first(g, ne, wt0, b0.reshape(1, -1), C)
        a, bb = _finalize_stats(s, q, R, g0, be0)
        for (W, b, gg, be) in layers[1:]:
            z, s, q = _conv_mid(z, a, bb, jnp.transpose(W), b.reshape(1, -1))
            a, bb = _finalize_stats(s, q, R, gg, be)
        outs.append(_pool_max(z, a, bb, K))
    return new_c, new_rows, jnp.concatenate(outs, axis=1)


def _fp_level(x1_c, p1_rows, x2_c, p2_rows, layer_params, materialize=True):
    B = x1_c.shape[0]
    N = x1_c.shape[2]
    S = x2_c.shape[2]
    R = B * N

    x1_rows3 = jnp.transpose(x1_c, (0, 2, 1))
    packed = _knn3_call(x1_rows3, x2_c)
    arr = packed.reshape(R, 8)
    i0 = arr[:, 0].astype(jnp.int32)
    i1 = arr[:, 1].astype(jnp.int32)
    i2 = arr[:, 2].astype(jnp.int32)
    recip = 1.0 / (arr[:, 3:6] + 1e-8)
    w3 = recip / jnp.sum(recip, axis=1, keepdims=True)

    g_all = _gather_rows(p2_rows, jnp.concatenate([i0, i1, i2]))
    g0, g1, g2 = g_all[:R], g_all[R:2 * R], g_all[2 * R:]

    W0, b0, gg0, be0 = layer_params[0]
    C2 = p2_rows.shape[1]
    if p1_rows is not None:
        C1 = p1_rows.shape[1]
        wpt = jnp.transpose(W0[:, :C1])
        wit = jnp.transpose(W0[:, C1:])
    else:
        wpt = None
        wit = jnp.transpose(W0)
    z, s, q = _conv_fp_first(p1_rows, g0, g1, g2, w3, wpt, wit, b0.reshape(1, -1))
    a, bb = _finalize_stats(s, q, R, gg0, be0)
    for (W, b, gg, be) in layer_params[1:]:
        z, s, q = _conv_mid(z, a, bb, jnp.transpose(W), b.reshape(1, -1))
        a, bb = _finalize_stats(s, q, R, gg, be)
    if materialize:
        return _bn_relu(z, a, bb)
    return z, a, bb


def kernel(xyz, colors, params):
    del colors
    B, _, N0 = xyz.shape

    mean = jnp.mean(xyz, axis=2, keepdims=True)
    std = jnp.std(xyz, axis=2, keepdims=True, ddof=1)
    std = jnp.where(std == 0, 1e-8, std)
    xyzn = (xyz - mean) / std
    l0_rows = jnp.transpose(xyzn, (0, 2, 1)).reshape(B * N0, 3)

    l1_c, l1_rows, l1_p = _sa_level(
        xyzn, l0_rows, 1024, [0.05, 0.1], [16, 32], params['sa1'])
    l2_c, l2_rows, l2_p = _sa_level(
        l1_c, l1_p, 256, [0.1, 0.2], [16, 32], params['sa2'])
    l3_c, l3_rows, l3_p = _sa_level(
        l2_c, l2_p, 64, [0.2, 0.4], [16, 32], params['sa3'])
    l4_c, l4_rows, l4_p = _sa_level(
        l3_c, l3_p, 16, [0.4, 0.8], [16, 32], params['sa4'])

    l3_p = _fp_level(l3_c, l3_p, l4_c, l4_p, params['fp4'])
    l2_p = _fp_level(l2_c, l2_p, l3_c, l3_p, params['fp3'])
    l1_p = _fp_level(l1_c, l1_p, l2_c, l2_p, params['fp2'])
    z, a, bb = _fp_level(xyzn, None, l1_c, l1_p, params['fp1'],
                         materialize=False)

    W1, b1, g1, be1 = params['conv1']
    z, s, q = _conv_mid(z, a, bb, jnp.transpose(W1), b1.reshape(1, -1))
    a, bb = _finalize_stats(s, q, B * N0, g1, be1)

    W2, b2 = params['conv2']
    out = _final_call(z, a, bb, jnp.transpose(W2), b2.reshape(1, -1))
    return out.reshape(B, N0, W2.shape[0])
